# baseline (speedup 1.0000x reference)
import jax
import jax.numpy as jnp
from jax.experimental import pallas as pl

_TN = 512
_K = 1024
_D = 256


def _dot(a, b, precision=jax.lax.Precision.HIGHEST):
    return jax.lax.dot_general(
        a, b, dimension_numbers=(((1,), (0,)), ((), ())),
        preferred_element_type=jnp.float32, precision=precision)


def _rq_kernel(x_ref, cb0t_ref, cb1t_ref, cb2t_ref, cb0_ref, cb1_ref,
               o0_ref, o1_ref, o2_ref):
    r = x_ref[...]
    iota = jax.lax.broadcasted_iota(jnp.int32, (r.shape[0], _K), 1)

    def layer(r, cbt_ref, cb_ref, o_ref):
        cbt = cbt_ref[...]
        cn = jnp.sum(cbt * cbt, axis=0, keepdims=True)
        g = _dot(r, cbt, precision=jax.lax.Precision.DEFAULT)
        s = cn - 2.0 * g
        m = jnp.min(s, axis=1, keepdims=True)
        idx = jnp.min(jnp.where(s == m, iota, _K), axis=1, keepdims=True)
        o_ref[...] = idx
        if cb_ref is None:
            return r
        onehot = (iota == idx).astype(jnp.float32)
        q = _dot(onehot, cb_ref[...])
        return r - q

    r = layer(r, cb0t_ref, cb0_ref, o0_ref)
    r = layer(r, cb1t_ref, cb1_ref, o1_ref)
    layer(r, cb2t_ref, None, o2_ref)


def kernel(x, cb0, cb1, cb2):
    n, d = x.shape
    full = pl.BlockSpec((d, _K), lambda i: (0, 0))
    outs = pl.pallas_call(
        _rq_kernel,
        grid=(n // _TN,),
        in_specs=[
            pl.BlockSpec((_TN, d), lambda i: (i, 0)),
            full, full, full,
            pl.BlockSpec((_K, d), lambda i: (0, 0)),
            pl.BlockSpec((_K, d), lambda i: (0, 0)),
        ],
        out_specs=[pl.BlockSpec((_TN, 1), lambda i: (i, 0))] * 3,
        out_shape=[jax.ShapeDtypeStruct((n, 1), jnp.int32)] * 3,
    )(x, cb0.T, cb1.T, cb2.T, cb0, cb1)
    return jnp.concatenate(outs, axis=1)


# speedup vs baseline: 1.4974x; 1.4974x over previous
import jax
import jax.numpy as jnp
from jax.experimental import pallas as pl

_TN = 512
_K = 1024
_D = 256


def _dot(a, b, precision=jax.lax.Precision.DEFAULT):
    return jax.lax.dot_general(
        a, b, dimension_numbers=(((1,), (0,)), ((), ())),
        preferred_element_type=jnp.float32, precision=precision)


def _rq_kernel(x_ref, cb0t_ref, cb1t_ref, cb2t_ref, cb0s_ref, cb1s_ref,
               o0_ref, o1_ref, o2_ref):
    r = x_ref[...]
    iota = jax.lax.broadcasted_iota(jnp.int32, (r.shape[0], _K), 1)

    def layer(r, cbt_ref, cbs_ref, o_ref):
        cbt = cbt_ref[...]
        cn = jnp.sum(cbt * cbt, axis=0, keepdims=True)
        g = _dot(r, cbt)
        s = cn - 2.0 * g
        m = jnp.min(s, axis=1, keepdims=True)
        idx = jnp.min(jnp.where(s == m, iota, _K), axis=1, keepdims=True)
        o_ref[...] = idx
        if cbs_ref is None:
            return r
        onehot = (iota == idx).astype(jnp.bfloat16)
        q = (_dot(onehot, cbs_ref[0])
             + _dot(onehot, cbs_ref[1])
             + _dot(onehot, cbs_ref[2]))
        return r - q

    r = layer(r, cb0t_ref, cb0s_ref, o0_ref)
    r = layer(r, cb1t_ref, cb1s_ref, o1_ref)
    layer(r, cb2t_ref, None, o2_ref)


def _split3(cb):
    hi = cb.astype(jnp.bfloat16)
    rem = cb - hi.astype(jnp.float32)
    mid = rem.astype(jnp.bfloat16)
    lo = (rem - mid.astype(jnp.float32)).astype(jnp.bfloat16)
    return jnp.stack([hi, mid, lo])


def kernel(x, cb0, cb1, cb2):
    n, d = x.shape
    full = pl.BlockSpec((d, _K), lambda i: (0, 0))
    split = pl.BlockSpec((3, _K, d), lambda i: (0, 0, 0))
    outs = pl.pallas_call(
        _rq_kernel,
        grid=(n // _TN,),
        in_specs=[
            pl.BlockSpec((_TN, d), lambda i: (i, 0)),
            full, full, full,
            split, split,
        ],
        out_specs=[pl.BlockSpec((_TN, 1), lambda i: (i, 0))] * 3,
        out_shape=[jax.ShapeDtypeStruct((n, 1), jnp.int32)] * 3,
    )(x, cb0.T, cb1.T, cb2.T, _split3(cb0), _split3(cb1))
    return jnp.concatenate(outs, axis=1)
